# baseline (speedup 1.0000x reference)
import functools

import jax
import jax.numpy as jnp
from jax import lax
from jax.experimental import pallas as pl
from jax.experimental.pallas import tpu as pltpu
from jax.experimental.pallas import tpu_sc as plsc

_EMB = 32
_ROW = 128
_IDXW = 128
_CHUNK = 256
_K = _CHUNK // _IDXW


@functools.lru_cache(maxsize=None)
def _build_gather(bln):
    nc, ns = 2, 16
    nw = nc * ns
    n_chunks = bln // _CHUNK
    chunks_per_w = n_chunks // nw
    assert bln % (nw * _CHUNK) == 0

    def body(node_hbm, trans_hbm, nid_hbm, tid_hbm, out_hbm,
             idx_n, idx_t, comb, sem):
        wid = lax.axis_index("s") * nc + lax.axis_index("c")
        c0 = wid * chunks_per_w

        def step(i, carry):
            ci = c0 + i
            pltpu.sync_copy(nid_hbm.at[ci], idx_n)
            pltpu.sync_copy(tid_hbm.at[ci], idx_t)
            cps = []
            for j in range(_K):
                dst = pl.ds(j * _IDXW, _IDXW)
                cps.append(pltpu.async_copy(
                    node_hbm.at[idx_n.at[j]], comb.at[dst, 0], sem))
                cps.append(pltpu.async_copy(
                    trans_hbm.at[idx_t.at[j]], comb.at[dst, 1], sem))
            for cp in cps:
                cp.wait()
            pltpu.sync_copy(comb, out_hbm.at[pl.ds(ci * _CHUNK, _CHUNK)])
            return carry

        lax.fori_loop(0, chunks_per_w, step, 0)

    return pl.kernel(
        body,
        mesh=plsc.VectorSubcoreMesh(core_axis_name="c", subcore_axis_name="s"),
        out_type=jax.ShapeDtypeStruct((bln, 2, _ROW), jnp.float32),
        scratch_types=[
            pltpu.VMEM((_K, _IDXW), jnp.int32),
            pltpu.VMEM((_K, _IDXW), jnp.int32),
            pltpu.VMEM((_CHUNK, 2, _ROW), jnp.float32),
            pltpu.SemaphoreType.DMA,
        ],
    )


def kernel(input_X, input_A, node_table, transition_table, max_batch_length):
    b, mbl, n, _ = input_X.shape
    bln = b * mbl * n
    delta = jnp.asarray(max_batch_length).astype(jnp.int32) - mbl
    flat_x = input_X.reshape(bln, input_X.shape[-1]).astype(jnp.int32)
    nid = (flat_x[:, 1] + delta).reshape(bln // _CHUNK, _K, _IDXW)
    tid = (flat_x[:, 4] + delta).reshape(bln // _CHUNK, _K, _IDXW)
    node_p = jnp.pad(node_table, ((0, 0), (0, _ROW - _EMB)))
    trans_p = jnp.pad(transition_table, ((0, 0), (0, _ROW - _EMB)))
    out = _build_gather(bln)(node_p, trans_p, nid, tid)
    emb = out[:, :, :_EMB].reshape(b, mbl, n, 2 * _EMB)
    return (emb, input_A.astype(jnp.float32))


# speedup vs baseline: 1.2967x; 1.2967x over previous
import functools

import jax
import jax.numpy as jnp
from jax import lax
from jax.experimental import pallas as pl
from jax.experimental.pallas import tpu as pltpu
from jax.experimental.pallas import tpu_sc as plsc

_EMB = 32
_ROW = 128
_IDXW = 128
_L = 16


@functools.lru_cache(maxsize=None)
def _build_gather(n_slab, batch):
    nc, ns = 2, 16
    nw = nc * ns
    slabs_per_w = n_slab // nw
    kb = batch // _IDXW
    assert n_slab % nw == 0 and batch % _IDXW == 0

    def body(node_hbm, trans_hbm, idx_hbm, out_hbm, idxb, gb_n, gb_t, comb, sem):
        wid = lax.axis_index("s") * nc + lax.axis_index("c")
        s0 = wid * slabs_per_w
        iota = lax.iota(jnp.int32, _L)

        def slab(si, carry):
            s = s0 + si
            pltpu.sync_copy(idx_hbm.at[s], idxb)
            cps = []
            for j in range(kb):
                dst = pl.ds(j * _IDXW, _IDXW)
                cps.append(pltpu.async_copy(
                    node_hbm.at[idxb.at[j]], gb_n.at[dst], sem))
                cps.append(pltpu.async_copy(
                    trans_hbm.at[idxb.at[kb + j]], gb_t.at[dst], sem))
            for cp in cps:
                cp.wait()

            def grp(gi, carry2):
                for t in range(2):
                    src = (gb_n, gb_t)[t]
                    for j in range(kb):
                        col0 = j * _IDXW + gi * _L
                        rows = col0 + iota
                        ids = idxb[(2 + t) * kb + j, pl.ds(gi * _L, _L)]
                        cols = (ids & 3) << 5
                        for e in range(_EMB):
                            v = plsc.load_gather(src, [rows, cols + e])
                            comb[t * _EMB + e, pl.ds(col0, _L)] = v
                return carry2

            lax.fori_loop(0, _IDXW // _L, grp, 0)
            pltpu.sync_copy(comb, out_hbm.at[s])
            return carry

        lax.fori_loop(0, slabs_per_w, slab, 0)

    return pl.kernel(
        body,
        mesh=plsc.VectorSubcoreMesh(core_axis_name="c", subcore_axis_name="s"),
        compiler_params=pltpu.CompilerParams(needs_layout_passes=False),
        out_type=jax.ShapeDtypeStruct((n_slab, 2 * _EMB, batch), jnp.float32),
        scratch_types=[
            pltpu.VMEM((4 * kb, _IDXW), jnp.int32),
            pltpu.VMEM((batch, _ROW), jnp.float32),
            pltpu.VMEM((batch, _ROW), jnp.float32),
            pltpu.VMEM((2 * _EMB, batch), jnp.float32),
            pltpu.SemaphoreType.DMA,
        ],
    )


def _pack(table):
    rows = (table.shape[0] * _EMB) // _ROW
    return table.reshape(-1)[: rows * _ROW].reshape(rows, _ROW)


def kernel(input_X, input_A, node_table, transition_table, max_batch_length):
    b, mbl, n, _ = input_X.shape
    n_slab = mbl * n
    kb = b // _IDXW
    delta = jnp.asarray(max_batch_length).astype(jnp.int32) - mbl
    flat_x = input_X.reshape(b, n_slab, input_X.shape[-1]).astype(jnp.int32)
    nid = (flat_x[:, :, 1] + delta).T.reshape(n_slab, kb, _IDXW)
    tid = (flat_x[:, :, 4] + delta).T.reshape(n_slab, kb, _IDXW)
    idx_all = jnp.concatenate(
        [nid >> 2, tid >> 2, nid, tid], axis=1)
    out = _build_gather(n_slab, b)(_pack(node_table), _pack(transition_table),
                                   idx_all, )
    emb = out.reshape(mbl, n, 2 * _EMB, b).transpose(3, 0, 1, 2)
    return (emb, input_A.astype(jnp.float32))


# speedup vs baseline: 1.7078x; 1.3171x over previous
import functools

import jax
import jax.numpy as jnp
from jax import lax
from jax.experimental import pallas as pl
from jax.experimental.pallas import tpu as pltpu
from jax.experimental.pallas import tpu_sc as plsc

_EMB = 32
_ROW = 128
_IDXW = 128
_L = 16


@functools.lru_cache(maxsize=None)
def _build_gather(n_slab, batch):
    nc, ns = 2, 16
    nw = nc * ns
    slabs_per_w = n_slab // nw
    kb = batch // _IDXW
    assert n_slab % nw == 0 and batch % _IDXW == 0

    def body(node_hbm, trans_hbm, idx_hbm, out_hbm, idxb, gb_n, gb_t, comb, sem):
        wid = lax.axis_index("s") * nc + lax.axis_index("c")
        s0 = wid * slabs_per_w
        iota = lax.iota(jnp.int32, _L)

        def slab(si, carry):
            s = s0 + si
            pltpu.sync_copy(idx_hbm.at[s], idxb)
            cps = []
            for j in range(kb):
                dst = pl.ds(j * _IDXW, _IDXW)
                cps.append(pltpu.async_copy(
                    node_hbm.at[idxb.at[j]], gb_n.at[dst], sem))
                cps.append(pltpu.async_copy(
                    trans_hbm.at[idxb.at[kb + j]], gb_t.at[dst], sem))
            for cp in cps:
                cp.wait()

            def grp(gi, carry2):
                pre = []
                for t in range(2):
                    for j in range(kb):
                        col0 = j * _IDXW + gi * _L
                        rows = col0 + iota
                        ids = idxb[(2 + t) * kb + j, pl.ds(gi * _L, _L)]
                        pre.append((t, (gb_n, gb_t)[t], rows,
                                    (ids & 3) << 5))
                for e0 in range(_EMB):
                    m = (e0 + iota) & (_EMB - 1)
                    for t, src, rows, q32 in pre:
                        v = plsc.load_gather(src, [rows, q32 + m])
                        plsc.store_scatter(comb, [m + t * _EMB, rows], v)
                return carry2

            lax.fori_loop(0, _IDXW // _L, grp, 0)
            pltpu.sync_copy(comb, out_hbm.at[s])
            return carry

        lax.fori_loop(0, slabs_per_w, slab, 0)

    return pl.kernel(
        body,
        mesh=plsc.VectorSubcoreMesh(core_axis_name="c", subcore_axis_name="s"),
        compiler_params=pltpu.CompilerParams(needs_layout_passes=False),
        out_type=jax.ShapeDtypeStruct((n_slab, 2 * _EMB, batch), jnp.float32),
        scratch_types=[
            pltpu.VMEM((4 * kb, _IDXW), jnp.int32),
            pltpu.VMEM((batch, _ROW), jnp.float32),
            pltpu.VMEM((batch, _ROW), jnp.float32),
            pltpu.VMEM((2 * _EMB, batch), jnp.float32),
            pltpu.SemaphoreType.DMA,
        ],
    )


def _pack(table):
    rows = (table.shape[0] * _EMB) // _ROW
    return table.reshape(-1)[: rows * _ROW].reshape(rows, _ROW)


def kernel(input_X, input_A, node_table, transition_table, max_batch_length):
    b, mbl, n, _ = input_X.shape
    n_slab = mbl * n
    kb = b // _IDXW
    delta = jnp.asarray(max_batch_length).astype(jnp.int32) - mbl
    flat_x = input_X.reshape(b, n_slab, input_X.shape[-1]).astype(jnp.int32)
    nid = (flat_x[:, :, 1] + delta).T.reshape(n_slab, kb, _IDXW)
    tid = (flat_x[:, :, 4] + delta).T.reshape(n_slab, kb, _IDXW)
    idx_all = jnp.concatenate(
        [nid >> 2, tid >> 2, nid, tid], axis=1)
    out = _build_gather(n_slab, b)(_pack(node_table), _pack(transition_table),
                                   idx_all, )
    emb = out.reshape(mbl, n, 2 * _EMB, b).transpose(3, 0, 1, 2)
    return (emb, input_A.astype(jnp.float32))


# speedup vs baseline: 1.9184x; 1.1233x over previous
import functools

import jax
import jax.numpy as jnp
from jax import lax
from jax.experimental import pallas as pl
from jax.experimental.pallas import tpu as pltpu
from jax.experimental.pallas import tpu_sc as plsc

_EMB = 32
_ROW = 128
_IDXW = 128
_L = 16


@functools.lru_cache(maxsize=None)
def _build_gather(n_slab, batch):
    nc, ns = 2, 16
    nw = nc * ns
    slabs_per_w = n_slab // nw
    kb = batch // _IDXW
    assert n_slab % (2 * nw) == 0 and kb == 2

    def body(node_hbm, trans_hbm, idx_hbm, out_hbm,
             idxb0, idxb1, gn0, gn1, gt0, gt1, comb0, comb1,
             semg0, semg1, semw0, semw1):
        idxb, gn, gt = (idxb0, idxb1), (gn0, gn1), (gt0, gt1)
        comb, semg, semw = (comb0, comb1), (semg0, semg1), (semw0, semw1)
        wid = lax.axis_index("s") * nc + lax.axis_index("c")
        s0 = wid * slabs_per_w
        iota = lax.iota(jnp.int32, _L)

        def fire(s, islot, h, gslot):
            pltpu.async_copy(node_hbm.at[idxb[islot].at[h]], gn[gslot],
                             semg[gslot])
            pltpu.async_copy(trans_hbm.at[idxb[islot].at[2 + h]], gt[gslot],
                             semg[gslot])

        def wait_g(islot, h, gslot):
            pltpu.make_async_copy(node_hbm.at[idxb[islot].at[h]], gn[gslot],
                                  semg[gslot]).wait()
            pltpu.make_async_copy(trans_hbm.at[idxb[islot].at[2 + h]],
                                  gt[gslot], semg[gslot]).wait()

        def assemble(islot, h, sp):
            def grp(gi, carry2):
                pre = []
                for t in range(2):
                    lrows = gi * _L + iota
                    ids = idxb[islot][4 + 2 * t + h, pl.ds(gi * _L, _L)]
                    pre.append((t, (gn, gt)[t][h], lrows, (ids & 3) << 5))
                ocol = h * _IDXW + gi * _L + iota
                for e0 in range(_EMB):
                    m = (e0 + iota) & (_EMB - 1)
                    for t, src, lrows, q32 in pre:
                        v = plsc.load_gather(src, [lrows, q32 + m])
                        plsc.store_scatter(comb[sp], [m + t * _EMB, ocol], v)
                return carry2

            lax.fori_loop(0, _IDXW // _L, grp, 0)

        pltpu.sync_copy(idx_hbm.at[s0], idxb[0])
        fire(s0, 0, 0, 0)

        def pair(gg, carry):
            for sp in (0, 1):
                sl = gg * 2 + sp
                s = s0 + sl

                @pl.when(sl < slabs_per_w - 1)
                def _():
                    pltpu.sync_copy(idx_hbm.at[s + 1], idxb[(sp + 1) % 2])

                @pl.when(sl >= 2)
                def _():
                    pltpu.make_async_copy(
                        comb[sp], out_hbm.at[s0 + sp], semw[sp]).wait()

                for h in (0, 1):
                    if h == 0:
                        fire(s, sp, 1, 1)
                    else:
                        @pl.when(sl < slabs_per_w - 1)
                        def _():
                            fire(s + 1, (sp + 1) % 2, 0, 0)
                    wait_g(sp, h, h)
                    assemble(sp, h, sp)

                pltpu.async_copy(comb[sp], out_hbm.at[s], semw[sp])
            return carry

        lax.fori_loop(0, slabs_per_w // 2, pair, 0)
        pltpu.make_async_copy(comb[0], out_hbm.at[s0], semw[0]).wait()
        pltpu.make_async_copy(comb[1], out_hbm.at[s0], semw[1]).wait()

    return pl.kernel(
        body,
        mesh=plsc.VectorSubcoreMesh(core_axis_name="c", subcore_axis_name="s"),
        compiler_params=pltpu.CompilerParams(needs_layout_passes=False),
        out_type=jax.ShapeDtypeStruct((n_slab, 2 * _EMB, batch), jnp.float32),
        scratch_types=[
            pltpu.VMEM((4 * 2, _IDXW), jnp.int32),
            pltpu.VMEM((4 * 2, _IDXW), jnp.int32),
            pltpu.VMEM((_IDXW, _ROW), jnp.float32),
            pltpu.VMEM((_IDXW, _ROW), jnp.float32),
            pltpu.VMEM((_IDXW, _ROW), jnp.float32),
            pltpu.VMEM((_IDXW, _ROW), jnp.float32),
            pltpu.VMEM((2 * _EMB, batch), jnp.float32),
            pltpu.VMEM((2 * _EMB, batch), jnp.float32),
            pltpu.SemaphoreType.DMA,
            pltpu.SemaphoreType.DMA,
            pltpu.SemaphoreType.DMA,
            pltpu.SemaphoreType.DMA,
        ],
    )


def _pack(table):
    rows = (table.shape[0] * _EMB) // _ROW
    return table.reshape(-1)[: rows * _ROW].reshape(rows, _ROW)


def kernel(input_X, input_A, node_table, transition_table, max_batch_length):
    b, mbl, n, _ = input_X.shape
    n_slab = mbl * n
    kb = b // _IDXW
    delta = jnp.asarray(max_batch_length).astype(jnp.int32) - mbl
    flat_x = input_X.reshape(b, n_slab, input_X.shape[-1]).astype(jnp.int32)
    nid = (flat_x[:, :, 1] + delta).T.reshape(n_slab, kb, _IDXW)
    tid = (flat_x[:, :, 4] + delta).T.reshape(n_slab, kb, _IDXW)
    idx_all = jnp.concatenate(
        [nid >> 2, tid >> 2, nid, tid], axis=1)
    out = _build_gather(n_slab, b)(_pack(node_table), _pack(transition_table),
                                   idx_all)
    emb = out.reshape(mbl, n, 2 * _EMB, b).transpose(3, 0, 1, 2)
    return (emb, input_A.astype(jnp.float32))


# speedup vs baseline: 2.0384x; 1.0626x over previous
import functools

import jax
import jax.numpy as jnp
from jax import lax
from jax.experimental import pallas as pl
from jax.experimental.pallas import tpu as pltpu
from jax.experimental.pallas import tpu_sc as plsc

_EMB = 32
_ROW = 128
_IDXW = 128
_L = 16


@functools.lru_cache(maxsize=None)
def _build_gather(n_slab, batch):
    nc, ns = 2, 16
    nw = nc * ns
    slabs_per_w = n_slab // nw
    kb = batch // _IDXW
    assert n_slab % (2 * nw) == 0 and kb == 2

    def body(node_hbm, trans_hbm, idx_hbm, out_hbm,
             idxb0, idxb1, gn0, gn1, gt0, gt1, comb0, comb1,
             semg0, semg1, semw0, semw1, semi0, semi1):
        idxb, gn, gt = (idxb0, idxb1), (gn0, gn1), (gt0, gt1)
        comb, semg, semw = (comb0, comb1), (semg0, semg1), (semw0, semw1)
        semi = (semi0, semi1)
        wid = lax.axis_index("s") * nc + lax.axis_index("c")
        s0 = wid * slabs_per_w
        iota = lax.iota(jnp.int32, _L)

        def fire(s, islot, h, gslot):
            pltpu.async_copy(node_hbm.at[idxb[islot].at[h]], gn[gslot],
                             semg[gslot])
            pltpu.async_copy(trans_hbm.at[idxb[islot].at[2 + h]], gt[gslot],
                             semg[gslot])

        def wait_g(islot, h, gslot, t):
            if t == 0:
                pltpu.make_async_copy(node_hbm.at[idxb[islot].at[h]],
                                      gn[gslot], semg[gslot]).wait()
            else:
                pltpu.make_async_copy(trans_hbm.at[idxb[islot].at[2 + h]],
                                      gt[gslot], semg[gslot]).wait()

        def assemble(islot, h, sp, t):
            def grp(gi, carry2):
                lrows = gi * _L + iota
                ids = idxb[islot][4 + 2 * t + h, pl.ds(gi * _L, _L)]
                q32 = (ids & 3) << 5
                src = (gn, gt)[t][h]
                ocol = h * _IDXW + gi * _L + iota
                for e0 in range(_EMB):
                    m = (e0 + iota) & (_EMB - 1)
                    v = plsc.load_gather(src, [lrows, q32 + m])
                    plsc.store_scatter(comb[sp], [m + t * _EMB, ocol], v)
                return carry2

            lax.fori_loop(0, _IDXW // _L, grp, 0)

        pltpu.sync_copy(idx_hbm.at[s0], idxb[0])
        fire(s0, 0, 0, 0)

        def pair(gg, carry):
            for sp in (0, 1):
                sl = gg * 2 + sp
                s = s0 + sl

                @pl.when(sl >= 1)
                def _():
                    pltpu.make_async_copy(
                        idx_hbm.at[s], idxb[sp], semi[sp]).wait()

                @pl.when(sl < slabs_per_w - 1)
                def _():
                    pltpu.async_copy(
                        idx_hbm.at[s + 1], idxb[(sp + 1) % 2],
                        semi[(sp + 1) % 2])

                @pl.when(sl >= 2)
                def _():
                    pltpu.make_async_copy(
                        comb[sp], out_hbm.at[s0 + sp], semw[sp]).wait()

                for h in (0, 1):
                    if h == 0:
                        fire(s, sp, 1, 1)
                    else:
                        @pl.when(sl < slabs_per_w - 1)
                        def _():
                            fire(s + 1, (sp + 1) % 2, 0, 0)
                    wait_g(sp, h, h, 0)
                    assemble(sp, h, sp, 0)
                    wait_g(sp, h, h, 1)
                    assemble(sp, h, sp, 1)

                pltpu.async_copy(comb[sp], out_hbm.at[s], semw[sp])
            return carry

        lax.fori_loop(0, slabs_per_w // 2, pair, 0)
        pltpu.make_async_copy(comb[0], out_hbm.at[s0], semw[0]).wait()
        pltpu.make_async_copy(comb[1], out_hbm.at[s0], semw[1]).wait()

    return pl.kernel(
        body,
        mesh=plsc.VectorSubcoreMesh(core_axis_name="c", subcore_axis_name="s"),
        compiler_params=pltpu.CompilerParams(needs_layout_passes=False),
        out_type=jax.ShapeDtypeStruct((n_slab, 2 * _EMB, batch), jnp.float32),
        scratch_types=[
            pltpu.VMEM((4 * 2, _IDXW), jnp.int32),
            pltpu.VMEM((4 * 2, _IDXW), jnp.int32),
            pltpu.VMEM((_IDXW, _ROW), jnp.float32),
            pltpu.VMEM((_IDXW, _ROW), jnp.float32),
            pltpu.VMEM((_IDXW, _ROW), jnp.float32),
            pltpu.VMEM((_IDXW, _ROW), jnp.float32),
            pltpu.VMEM((2 * _EMB, batch), jnp.float32),
            pltpu.VMEM((2 * _EMB, batch), jnp.float32),
            pltpu.SemaphoreType.DMA,
            pltpu.SemaphoreType.DMA,
            pltpu.SemaphoreType.DMA,
            pltpu.SemaphoreType.DMA,
            pltpu.SemaphoreType.DMA,
            pltpu.SemaphoreType.DMA,
        ],
    )


def _pack(table):
    rows = (table.shape[0] * _EMB) // _ROW
    return table.reshape(-1)[: rows * _ROW].reshape(rows, _ROW)


def kernel(input_X, input_A, node_table, transition_table, max_batch_length):
    b, mbl, n, _ = input_X.shape
    n_slab = mbl * n
    kb = b // _IDXW
    delta = jnp.asarray(max_batch_length).astype(jnp.int32) - mbl
    flat_x = input_X.reshape(b, n_slab, input_X.shape[-1]).astype(jnp.int32)
    nid = (flat_x[:, :, 1] + delta).T.reshape(n_slab, kb, _IDXW)
    tid = (flat_x[:, :, 4] + delta).T.reshape(n_slab, kb, _IDXW)
    idx_all = jnp.concatenate(
        [nid >> 2, tid >> 2, nid, tid], axis=1)
    out = _build_gather(n_slab, b)(_pack(node_table), _pack(transition_table),
                                   idx_all)
    emb = out.reshape(mbl, n, 2 * _EMB, b).transpose(3, 0, 1, 2)
    return (emb, input_A.astype(jnp.float32))
